# baseline (speedup 1.0000x reference)
import functools

import jax
import jax.numpy as jnp
from jax import lax
from jax.experimental import pallas as pl
from jax.experimental.pallas import tpu as pltpu
from jax.experimental.pallas import tpu_sc as plsc

_NUM_FIELDS = 26
_VOCAB = 100000
_EMB = 32
_BATCH = 16384
_NW = 32
_CH = 2048
_NCHK = _BATCH // _CH
_LANES = 16
_GROUPS = (7, 7, 6, 6)

_mesh = plsc.VectorSubcoreMesh(core_axis_name="c", subcore_axis_name="s")


@functools.cache
def _make_group_kernel(nf):
    @functools.partial(
        pl.kernel,
        mesh=_mesh,
        out_type=jax.ShapeDtypeStruct((nf * _EMB, _BATCH), jnp.float32),
        scratch_types=[
            pltpu.VMEM((_VOCAB,), jnp.float32),
            pltpu.VMEM((_BATCH,), jnp.int32),
            pltpu.VMEM((2, _CH), jnp.float32),
            pltpu.SemaphoreType.DMA,
            pltpu.SemaphoreType.DMA,
            pltpu.SemaphoreType.DMA,
        ],
        compiler_params=pltpu.CompilerParams(
            use_tc_tiling_on_sc=False, needs_layout_passes=False),
    )
    def _sc_embed(xt_hbm, tablest_hbm, out_hbm, vec, xrow, ob,
                  sem_v, sem_x, sem_st):
        wid = lax.axis_index("s") * 2 + lax.axis_index("c")

        for j in range(nf):
            pltpu.async_copy(xt_hbm.at[j], xrow, sem_x)
            pltpu.async_copy(tablest_hbm.at[j, wid], vec, sem_v)
            pltpu.make_async_copy(xt_hbm.at[j], xrow, sem_x).wait()
            pltpu.make_async_copy(tablest_hbm.at[j, wid], vec, sem_v).wait()
            row = j * _EMB + wid

            sd = {}
            for k in range(_NCHK):
                buf = k % 2
                if k >= 2:
                    sd.pop(k - 2).wait()

                def gstep(t, carry):
                    idx = xrow[pl.ds(k * _CH + t * _LANES, _LANES)]
                    ob[buf, pl.ds(t * _LANES, _LANES)] = plsc.load_gather(
                        vec, [idx])
                    return carry

                lax.fori_loop(0, _CH // _LANES, gstep, 0, unroll=8)
                sd[k] = pltpu.async_copy(
                    ob.at[buf], out_hbm.at[row, pl.ds(k * _CH, _CH)], sem_st)
            sd.pop(_NCHK - 2).wait()
            sd.pop(_NCHK - 1).wait()

    return _sc_embed


def kernel(x_cat, tables):
    xt = x_cat.T
    tt = tables.transpose(0, 2, 1)
    parts = []
    f0 = 0
    for nf in _GROUPS:
        xt_g = lax.slice_in_dim(xt, f0, f0 + nf, axis=0)
        tt_g = lax.slice_in_dim(tt, f0, f0 + nf, axis=0)
        parts.append(_make_group_kernel(nf)(xt_g, tt_g))
        f0 += nf
    out_t = jnp.concatenate(parts, axis=0)
    return out_t.T


# speedup vs baseline: 1.0237x; 1.0237x over previous
import functools

import jax
import jax.numpy as jnp
from jax import lax
from jax.experimental import pallas as pl
from jax.experimental.pallas import tpu as pltpu
from jax.experimental.pallas import tpu_sc as plsc

_NUM_FIELDS = 26
_VOCAB = 100000
_EMB = 32
_BATCH = 16384
_NW = 32
_CH = 2048
_NCHK = _BATCH // _CH
_LANES = 16
_GROUPS = (7, 7, 6, 6)
_NBT = _BATCH // 128
_CHT = _CH // 128

_mesh = plsc.VectorSubcoreMesh(core_axis_name="c", subcore_axis_name="s")


@functools.cache
def _make_group_kernel(nf):
    @functools.partial(
        pl.kernel,
        mesh=_mesh,
        out_type=jax.ShapeDtypeStruct((nf * _EMB // 8, _NBT, 8, 128),
                                      jnp.float32),
        scratch_types=[
            pltpu.VMEM((_VOCAB,), jnp.float32),
            pltpu.VMEM((_BATCH,), jnp.int32),
            pltpu.VMEM((2, _CHT, 128), jnp.float32),
            pltpu.SemaphoreType.DMA,
            pltpu.SemaphoreType.DMA,
            pltpu.SemaphoreType.DMA,
        ],
        compiler_params=pltpu.CompilerParams(
            use_tc_tiling_on_sc=False, needs_layout_passes=False),
    )
    def _sc_embed(xt_hbm, tablest_hbm, out_hbm, vec, xrow, ob,
                  sem_v, sem_x, sem_st):
        wid = lax.axis_index("s") * 2 + lax.axis_index("c")

        for j in range(nf):
            pltpu.async_copy(xt_hbm.at[j], xrow, sem_x)
            pltpu.async_copy(tablest_hbm.at[j, wid], vec, sem_v)
            pltpu.make_async_copy(xt_hbm.at[j], xrow, sem_x).wait()
            pltpu.make_async_copy(tablest_hbm.at[j, wid], vec, sem_v).wait()
            row = j * _EMB + wid
            tr = row // 8
            sub = lax.rem(row, 8)

            sd = {}
            for k in range(_NCHK):
                buf = k % 2
                if k >= 2:
                    sd.pop(k - 2).wait()

                def gstep(t, carry):
                    idx = xrow[pl.ds(k * _CH + t * _LANES, _LANES)]
                    ob[buf, t // 8, pl.ds(lax.rem(t, 8) * _LANES, _LANES)] = (
                        plsc.load_gather(vec, [idx]))
                    return carry

                lax.fori_loop(0, _CH // _LANES, gstep, 0, unroll=8)
                sd[k] = pltpu.async_copy(
                    ob.at[buf],
                    out_hbm.at[tr, pl.ds(k * _CHT, _CHT), sub],
                    sem_st)
            sd.pop(_NCHK - 2).wait()
            sd.pop(_NCHK - 1).wait()

    return _sc_embed


def kernel(x_cat, tables):
    xt = x_cat.T
    tt = tables.transpose(0, 2, 1)
    parts = []
    f0 = 0
    for nf in _GROUPS:
        xt_g = lax.slice_in_dim(xt, f0, f0 + nf, axis=0)
        tt_g = lax.slice_in_dim(tt, f0, f0 + nf, axis=0)
        parts.append(_make_group_kernel(nf)(xt_g, tt_g))
        f0 += nf
    out4d = jnp.concatenate(parts, axis=0)
    return out4d.transpose(1, 3, 0, 2).reshape(_BATCH, _NUM_FIELDS * _EMB)


# speedup vs baseline: 2.4330x; 2.3767x over previous
import functools

import jax
import jax.numpy as jnp
from jax import lax
from jax.experimental import pallas as pl
from jax.experimental.pallas import tpu as pltpu
from jax.experimental.pallas import tpu_sc as plsc

_NUM_FIELDS = 26
_VOCAB = 100000
_EMB = 32
_BATCH = 16384
_NCOMP = _NUM_FIELDS * _EMB
_NW = 32
_CPW = _NCOMP // _NW
_CH = 2048
_NCHK = _BATCH // _CH
_LANES = 16

_mesh = plsc.VectorSubcoreMesh(core_axis_name="c", subcore_axis_name="s")


@functools.partial(
    pl.kernel,
    mesh=_mesh,
    out_type=jax.ShapeDtypeStruct((_NCOMP, _BATCH), jnp.float32),
    scratch_types=[
        pltpu.VMEM((_VOCAB,), jnp.float32),
        pltpu.VMEM((_BATCH,), jnp.int32),
        pltpu.VMEM((2, _CH), jnp.float32),
        pltpu.SemaphoreType.DMA,
        pltpu.SemaphoreType.DMA,
        pltpu.SemaphoreType.DMA,
    ],
    compiler_params=pltpu.CompilerParams(
        use_tc_tiling_on_sc=True, needs_layout_passes=False),
)
def _sc_embed(xt_hbm, tablest_hbm, out_hbm, vec, xrow, ob,
              sem_v, sem_x, sem_st):
    wid = lax.axis_index("s") * 2 + lax.axis_index("c")
    c0 = wid * _CPW

    def component(j, carry):
        cc = c0 + j
        fld = cc // _EMB
        comp = lax.rem(cc, _EMB)
        prev_fld = (cc - 1) // _EMB

        @pl.when(jnp.logical_or(j == 0, fld != prev_fld))
        def _():
            pltpu.async_copy(xt_hbm.at[fld], xrow, sem_x).wait()

        pltpu.async_copy(tablest_hbm.at[fld, comp], vec, sem_v).wait()

        for k in range(_NCHK):
            buf = k % 2
            if k >= 2:
                pltpu.make_async_copy(
                    ob.at[(k - 2) % 2],
                    out_hbm.at[cc, pl.ds((k - 2) * _CH, _CH)],
                    sem_st).wait()

            def gstep(t, carry2):
                idx = xrow[pl.ds(k * _CH + t * _LANES, _LANES)]
                ob[buf, pl.ds(t * _LANES, _LANES)] = plsc.load_gather(
                    vec, [idx])
                return carry2

            lax.fori_loop(0, _CH // _LANES, gstep, 0, unroll=8)
            pltpu.async_copy(
                ob.at[buf], out_hbm.at[cc, pl.ds(k * _CH, _CH)], sem_st)
        for k in (_NCHK - 2, _NCHK - 1):
            pltpu.make_async_copy(
                ob.at[k % 2],
                out_hbm.at[cc, pl.ds(k * _CH, _CH)],
                sem_st).wait()
        return carry

    lax.fori_loop(0, _CPW, component, 0)


def kernel(x_cat, tables):
    out_t = _sc_embed(x_cat.T, tables.transpose(0, 2, 1))
    return out_t.T
